# baseline (speedup 1.0000x reference)
import functools

import jax
import jax.numpy as jnp
from jax import lax
from jax.experimental import pallas as pl
from jax.experimental.pallas import tpu as pltpu
from jax.experimental.pallas import tpu_sc as plsc

T, D, E, F = 2048, 768, 8, 3072
TT = 256
TR = 512
R_MAX = 2 * T + E * TR
NT_MAX = R_MAX // TR
NTILES_PAD = 48
NW = 32
TPW = T // NW


def _gate_body(x_ref, wg_ref, r1_ref, r2_ref, w1_ref, w2_ref, tmap_ref,
               carry_ref, e1s, e2s, p1s, p2s):
    t = pl.program_id(0)
    nt = pl.num_programs(0) - 1

    @pl.when(t == 0)
    def _init():
        carry_ref[...] = jnp.zeros_like(carry_ref)

    @pl.when(t < nt)
    def _gate_tile():
        logits = jnp.dot(x_ref[...], wg_ref[...],
                         preferred_element_type=jnp.float32)
        m = jnp.max(logits, axis=1, keepdims=True)
        ex = jnp.exp(logits - m)
        probs = ex / jnp.sum(ex, axis=1, keepdims=True)
        idx = jax.lax.broadcasted_iota(jnp.int32, probs.shape, 1)
        v1 = jnp.max(probs, axis=1, keepdims=True)
        i1 = jnp.min(jnp.where(probs == v1, idx, E), axis=1, keepdims=True)
        mask1 = idx == i1
        probs2 = jnp.where(mask1, -jnp.inf, probs)
        v2 = jnp.max(probs2, axis=1, keepdims=True)
        i2 = jnp.min(jnp.where(jnp.logical_and(probs2 == v2, ~mask1), idx, E),
                     axis=1, keepdims=True)
        den = v1 + v2 + 1e-9

        idx16 = jax.lax.broadcasted_iota(jnp.int32, (TT, 16), 1)
        onehot = jnp.where(jnp.logical_or(idx16 == i1, idx16 == i2), 1.0, 0.0)
        rows = jax.lax.broadcasted_iota(jnp.int32, (TT, TT), 0)
        cols = jax.lax.broadcasted_iota(jnp.int32, (TT, TT), 1)
        tri = jnp.where(rows > cols, 1.0, 0.0)
        excl = jnp.dot(tri, onehot, preferred_element_type=jnp.float32)
        posd = excl + carry_ref[...]
        carry_ref[...] += jnp.sum(onehot, axis=0, keepdims=True)

        sl = pl.ds(t * TT, TT)
        e1s[sl, :] = i1
        e2s[sl, :] = i2
        p1s[sl, :] = jnp.sum(
            jnp.where(idx16 == i1, posd, 0.0), axis=1, keepdims=True
        ).astype(jnp.int32)
        p2s[sl, :] = jnp.sum(
            jnp.where(idx16 == i2, posd, 0.0), axis=1, keepdims=True
        ).astype(jnp.int32)
        w1_ref[sl, :] = v1 / den
        w2_ref[sl, :] = v2 / den

    @pl.when(t == nt)
    def _route():
        cnt = carry_ref[...].astype(jnp.int32)
        cpad = (cnt + (TR - 1)) & jnp.int32(~(TR - 1))
        ri = jax.lax.broadcasted_iota(jnp.int32, (16, 16), 0)
        ci = jax.lax.broadcasted_iota(jnp.int32, (16, 16), 1)
        triu = jnp.where(ri < ci, 1.0, 0.0)
        base16 = jnp.dot(cpad.astype(jnp.float32), triu,
                         preferred_element_type=jnp.float32).astype(jnp.int32)

        e1 = e1s[...]
        e2 = e2s[...]
        add1 = jnp.zeros_like(e1)
        add2 = jnp.zeros_like(e2)
        for e in range(E):
            be = base16[0, e]
            add1 += jnp.where(e1 == e, be, 0)
            add2 += jnp.where(e2 == e, be, 0)
        r1_ref[...] = p1s[...] + add1
        r2_ref[...] = p2s[...] + add2

        total = base16[0, E]
        tstart = jax.lax.broadcasted_iota(jnp.int32, (1, NTILES_PAD), 1) * TR
        acc = jnp.zeros((1, NTILES_PAD), jnp.int32)
        for e in range(E):
            acc += jnp.where(tstart >= base16[0, e], 1, 0)
        eidt = jnp.where(tstart < total, acc - 1, E)

        rank16 = jnp.dot(jnp.where(cnt > 0, 1.0, 0.0), triu,
                         preferred_element_type=jnp.float32).astype(jnp.int32)
        slot_t = jnp.zeros_like(eidt)
        first_t = jnp.zeros_like(eidt)
        for e in range(E):
            is_e = eidt == e
            slot_t += jnp.where(is_e, rank16[0, e] & 1, 0)
            first_t = jnp.where(
                jnp.logical_and(is_e, tstart == base16[0, e]), 1, first_t)
        nxt_t = jnp.full_like(eidt, E)
        for e in range(E - 1, -1, -1):
            has = cnt[0, e] > 0
            nxt_t = jnp.where(jnp.logical_and(eidt < e, has), e, nxt_t)
        z = jnp.zeros_like(eidt)
        tmap_ref[...] = jnp.concatenate(
            [eidt, slot_t, first_t, nxt_t, z, z, z, z], axis=0)


def _gate(x, Wg):
    nt = T // TT
    return pl.pallas_call(
        _gate_body,
        grid=(nt + 1,),
        in_specs=[
            pl.BlockSpec((TT, D), lambda t: (jnp.minimum(t, nt - 1), 0)),
            pl.BlockSpec((D, E), lambda t: (0, 0)),
        ],
        out_specs=[
            pl.BlockSpec((T, 1), lambda t: (0, 0)),
            pl.BlockSpec((T, 1), lambda t: (0, 0)),
            pl.BlockSpec((T, 1), lambda t: (0, 0)),
            pl.BlockSpec((T, 1), lambda t: (0, 0)),
            pl.BlockSpec((8, NTILES_PAD), lambda t: (0, 0)),
        ],
        out_shape=[
            jax.ShapeDtypeStruct((T, 1), jnp.int32),
            jax.ShapeDtypeStruct((T, 1), jnp.int32),
            jax.ShapeDtypeStruct((T, 1), jnp.float32),
            jax.ShapeDtypeStruct((T, 1), jnp.float32),
            jax.ShapeDtypeStruct((8, NTILES_PAD), jnp.int32),
        ],
        scratch_shapes=[
            pltpu.VMEM((1, 16), jnp.float32),
            pltpu.VMEM((T, 1), jnp.int32),
            pltpu.VMEM((T, 1), jnp.int32),
            pltpu.VMEM((T, 1), jnp.int32),
            pltpu.VMEM((T, 1), jnp.int32),
        ],
    )(x, Wg)


def _dispatch_body(x_hbm, r1_hbm, r2_hbm, xd_hbm,
                   rows_v, r1v, r2v, sem):
    wid = lax.axis_index("s") * 2 + lax.axis_index("c")
    base_t = wid * TPW
    pltpu.sync_copy(r1_hbm.at[pl.ds(base_t, TPW)], r1v)
    pltpu.sync_copy(r2_hbm.at[pl.ds(base_t, TPW)], r2v)
    pltpu.sync_copy(x_hbm.at[pl.ds(base_t, TPW)], rows_v)
    pltpu.async_copy(rows_v, xd_hbm.at[r1v], sem).wait()
    pltpu.async_copy(rows_v, xd_hbm.at[r2v], sem).wait()


def _dispatch(x, r1, r2):
    mesh = plsc.VectorSubcoreMesh(core_axis_name="c", subcore_axis_name="s")
    fn = pl.kernel(
        _dispatch_body,
        out_type=[
            jax.ShapeDtypeStruct((R_MAX, D), jnp.float32),
        ],
        mesh=mesh,
        scratch_types=[
            pltpu.VMEM((TPW, D), jnp.float32),
            pltpu.VMEM((TPW,), jnp.int32),
            pltpu.VMEM((TPW,), jnp.int32),
            pltpu.SemaphoreType.DMA,
        ],
    )
    return fn(x, r1, r2)[0]


def _gmm_body(tmap_ref, xd_ref, w1_any, w2_any, b1_ref, b2_ref, yd_ref,
              w1b, w2b, s1, s2):
    i = pl.program_id(0)
    e = tmap_ref[0, i]
    slot = tmap_ref[1, i]
    first = tmap_ref[2, i]
    nxt = tmap_ref[3, i]

    NC = 4
    CD, CF = D // NC, F // NC

    def _chunks(dst, expert):
        for c in range(NC):
            yield pltpu.make_async_copy(
                w1_any.at[expert, pl.ds(c * CD, CD), :],
                w1b.at[dst, pl.ds(c * CD, CD), :], s1.at[dst])
            yield pltpu.make_async_copy(
                w2_any.at[expert, pl.ds(c * CF, CF), :],
                w2b.at[dst, pl.ds(c * CF, CF), :], s2.at[dst])

    def _issue(dst, expert):
        for cp in _chunks(dst, expert):
            cp.start()

    def _wait(dst, expert):
        for cp in _chunks(dst, expert):
            cp.wait()

    @pl.when(i == 0)
    def _cold():
        _issue(0, e)

    @pl.when(first == 1)
    def _boundary():
        @pl.when(jnp.logical_and(nxt < E, slot == 0))
        def _pf1():
            _issue(1, nxt)

        @pl.when(jnp.logical_and(nxt < E, slot == 1))
        def _pf0():
            _issue(0, nxt)

        @pl.when(slot == 0)
        def _w0():
            _wait(0, e)

        @pl.when(slot == 1)
        def _w1():
            _wait(1, e)

    def _compute(s):
        h = jnp.dot(xd_ref[...].astype(jnp.bfloat16),
                    w1b[s].astype(jnp.bfloat16),
                    preferred_element_type=jnp.float32)
        h = jax.nn.gelu(h + b1_ref[0])
        yd_ref[...] = jnp.dot(h.astype(jnp.bfloat16),
                              w2b[s].astype(jnp.bfloat16),
                              preferred_element_type=jnp.float32) + b2_ref[0]

    @pl.when(jnp.logical_and(e < E, slot == 0))
    def _c0():
        _compute(0)

    @pl.when(jnp.logical_and(e < E, slot == 1))
    def _c1():
        _compute(1)


def _gmm(tmap, Xd, W1, b1, W2, b2):
    def bmap(i, tm):
        return (jnp.minimum(tm[0, i], E - 1), 0, 0)

    grid_spec = pltpu.PrefetchScalarGridSpec(
        num_scalar_prefetch=1,
        grid=(NT_MAX,),
        in_specs=[
            pl.BlockSpec((TR, D), lambda i, tm: (i, 0)),
            pl.BlockSpec(memory_space=pltpu.MemorySpace.HBM),
            pl.BlockSpec(memory_space=pltpu.MemorySpace.HBM),
            pl.BlockSpec((1, 1, F), bmap),
            pl.BlockSpec((1, 1, D), bmap),
        ],
        out_specs=pl.BlockSpec((TR, D), lambda i, tm: (i, 0)),
        scratch_shapes=[
            pltpu.VMEM((2, D, F), jnp.float32),
            pltpu.VMEM((2, F, D), jnp.float32),
            pltpu.SemaphoreType.DMA((2,)),
            pltpu.SemaphoreType.DMA((2,)),
        ],
    )
    return pl.pallas_call(
        _gmm_body,
        grid_spec=grid_spec,
        out_shape=jax.ShapeDtypeStruct((R_MAX, D), jnp.float32),
        compiler_params=pltpu.CompilerParams(
            dimension_semantics=("arbitrary",),
        ),
    )(tmap, Xd, W1, W2, b1.reshape(E, 1, F), b2.reshape(E, 1, D))


def _gather2_body(yd_hbm, r1_hbm, r2_hbm, y1_hbm, y2_hbm,
                  r1v, r2v, a_v, b_v, sem1, sem2):
    wid = lax.axis_index("s") * 2 + lax.axis_index("c")
    base_t = wid * TPW
    pltpu.sync_copy(r1_hbm.at[pl.ds(base_t, TPW)], r1v)
    pltpu.sync_copy(r2_hbm.at[pl.ds(base_t, TPW)], r2v)
    cp1 = pltpu.async_copy(yd_hbm.at[r1v], a_v, sem1)
    cp2 = pltpu.async_copy(yd_hbm.at[r2v], b_v, sem2)
    cp1.wait()
    cp2.wait()
    pltpu.sync_copy(a_v, y1_hbm.at[pl.ds(base_t, TPW)])
    pltpu.sync_copy(b_v, y2_hbm.at[pl.ds(base_t, TPW)])


def _gather2(Yd, r1, r2):
    mesh = plsc.VectorSubcoreMesh(core_axis_name="c", subcore_axis_name="s")
    fn = pl.kernel(
        _gather2_body,
        out_type=[
            jax.ShapeDtypeStruct((T, D), jnp.float32),
            jax.ShapeDtypeStruct((T, D), jnp.float32),
        ],
        mesh=mesh,
        scratch_types=[
            pltpu.VMEM((TPW,), jnp.int32),
            pltpu.VMEM((TPW,), jnp.int32),
            pltpu.VMEM((TPW, D), jnp.float32),
            pltpu.VMEM((TPW, D), jnp.float32),
            pltpu.SemaphoreType.DMA,
            pltpu.SemaphoreType.DMA,
        ],
    )
    return fn(Yd, r1, r2)


def _fma_body(y1_ref, y2_ref, w1_ref, w2_ref, y_ref):
    y_ref[...] = w1_ref[...] * y1_ref[...] + w2_ref[...] * y2_ref[...]


def _combine(Y1, Y2, w1, w2):
    return pl.pallas_call(
        _fma_body,
        grid=(T // TT,),
        in_specs=[
            pl.BlockSpec((TT, D), lambda t: (t, 0)),
            pl.BlockSpec((TT, D), lambda t: (t, 0)),
            pl.BlockSpec((TT, 1), lambda t: (t, 0)),
            pl.BlockSpec((TT, 1), lambda t: (t, 0)),
        ],
        out_specs=pl.BlockSpec((TT, D), lambda t: (t, 0)),
        out_shape=jax.ShapeDtypeStruct((T, D), jnp.float32),
    )(Y1, Y2, w1, w2)


def kernel(x, Wg, W1, b1, W2, b2):
    r1, r2, w1, w2, tmap = _gate(x, Wg)
    r1 = r1.reshape(T)
    r2 = r2.reshape(T)
    Xd = _dispatch(x, r1, r2)
    Yd = _gmm(tmap, Xd, W1, b1, W2, b2)
    Y1, Y2 = _gather2(Yd, r1, r2)
    return _combine(Y1, Y2, w1, w2)


# speedup vs baseline: 1.0439x; 1.0439x over previous
import functools

import jax
import jax.numpy as jnp
from jax import lax
from jax.experimental import pallas as pl
from jax.experimental.pallas import tpu as pltpu
from jax.experimental.pallas import tpu_sc as plsc

T, D, E, F = 2048, 768, 8, 3072
TT = 256
TR = 256
R_MAX = 2 * T + E * TR
NT_MAX = R_MAX // TR
NTILES_PAD = 48
NW = 32
TPW = T // NW


def _gate_body(x_ref, wg_ref, r1_ref, r2_ref, w1_ref, w2_ref, tmap_ref,
               carry_ref, e1s, e2s, p1s, p2s):
    t = pl.program_id(0)
    nt = pl.num_programs(0) - 1

    @pl.when(t == 0)
    def _init():
        carry_ref[...] = jnp.zeros_like(carry_ref)

    @pl.when(t < nt)
    def _gate_tile():
        logits = jnp.dot(x_ref[...], wg_ref[...],
                         preferred_element_type=jnp.float32)
        m = jnp.max(logits, axis=1, keepdims=True)
        ex = jnp.exp(logits - m)
        probs = ex / jnp.sum(ex, axis=1, keepdims=True)
        idx = jax.lax.broadcasted_iota(jnp.int32, probs.shape, 1)
        v1 = jnp.max(probs, axis=1, keepdims=True)
        i1 = jnp.min(jnp.where(probs == v1, idx, E), axis=1, keepdims=True)
        mask1 = idx == i1
        probs2 = jnp.where(mask1, -jnp.inf, probs)
        v2 = jnp.max(probs2, axis=1, keepdims=True)
        i2 = jnp.min(jnp.where(jnp.logical_and(probs2 == v2, ~mask1), idx, E),
                     axis=1, keepdims=True)
        den = v1 + v2 + 1e-9

        idx16 = jax.lax.broadcasted_iota(jnp.int32, (TT, 16), 1)
        onehot = jnp.where(jnp.logical_or(idx16 == i1, idx16 == i2), 1.0, 0.0)
        rows = jax.lax.broadcasted_iota(jnp.int32, (TT, TT), 0)
        cols = jax.lax.broadcasted_iota(jnp.int32, (TT, TT), 1)
        tri = jnp.where(rows > cols, 1.0, 0.0)
        excl = jnp.dot(tri, onehot, preferred_element_type=jnp.float32)
        posd = excl + carry_ref[...]
        carry_ref[...] += jnp.sum(onehot, axis=0, keepdims=True)

        sl = pl.ds(t * TT, TT)
        e1s[sl, :] = i1
        e2s[sl, :] = i2
        p1s[sl, :] = jnp.sum(
            jnp.where(idx16 == i1, posd, 0.0), axis=1, keepdims=True
        ).astype(jnp.int32)
        p2s[sl, :] = jnp.sum(
            jnp.where(idx16 == i2, posd, 0.0), axis=1, keepdims=True
        ).astype(jnp.int32)
        w1_ref[sl, :] = v1 / den
        w2_ref[sl, :] = v2 / den

    @pl.when(t == nt)
    def _route():
        cnt = carry_ref[...].astype(jnp.int32)
        cpad = (cnt + (TR - 1)) & jnp.int32(~(TR - 1))
        ri = jax.lax.broadcasted_iota(jnp.int32, (16, 16), 0)
        ci = jax.lax.broadcasted_iota(jnp.int32, (16, 16), 1)
        triu = jnp.where(ri < ci, 1.0, 0.0)
        base16 = jnp.dot(cpad.astype(jnp.float32), triu,
                         preferred_element_type=jnp.float32).astype(jnp.int32)

        e1 = e1s[...]
        e2 = e2s[...]
        add1 = jnp.zeros_like(e1)
        add2 = jnp.zeros_like(e2)
        for e in range(E):
            be = base16[0, e]
            add1 += jnp.where(e1 == e, be, 0)
            add2 += jnp.where(e2 == e, be, 0)
        r1_ref[...] = p1s[...] + add1
        r2_ref[...] = p2s[...] + add2

        total = base16[0, E]
        tstart = jax.lax.broadcasted_iota(jnp.int32, (1, NTILES_PAD), 1) * TR
        acc = jnp.zeros((1, NTILES_PAD), jnp.int32)
        for e in range(E):
            acc += jnp.where(tstart >= base16[0, e], 1, 0)
        eidt = jnp.where(tstart < total, acc - 1, E)

        rank16 = jnp.dot(jnp.where(cnt > 0, 1.0, 0.0), triu,
                         preferred_element_type=jnp.float32).astype(jnp.int32)
        slot_t = jnp.zeros_like(eidt)
        first_t = jnp.zeros_like(eidt)
        for e in range(E):
            is_e = eidt == e
            slot_t += jnp.where(is_e, rank16[0, e] & 1, 0)
            first_t = jnp.where(
                jnp.logical_and(is_e, tstart == base16[0, e]), 1, first_t)
        nxt_t = jnp.full_like(eidt, E)
        for e in range(E - 1, -1, -1):
            has = cnt[0, e] > 0
            nxt_t = jnp.where(jnp.logical_and(eidt < e, has), e, nxt_t)
        z = jnp.zeros_like(eidt)
        tmap_ref[...] = jnp.concatenate(
            [eidt, slot_t, first_t, nxt_t, z, z, z, z], axis=0)


def _gate(x, Wg):
    nt = T // TT
    return pl.pallas_call(
        _gate_body,
        grid=(nt + 1,),
        in_specs=[
            pl.BlockSpec((TT, D), lambda t: (jnp.minimum(t, nt - 1), 0)),
            pl.BlockSpec((D, E), lambda t: (0, 0)),
        ],
        out_specs=[
            pl.BlockSpec((T, 1), lambda t: (0, 0)),
            pl.BlockSpec((T, 1), lambda t: (0, 0)),
            pl.BlockSpec((T, 1), lambda t: (0, 0)),
            pl.BlockSpec((T, 1), lambda t: (0, 0)),
            pl.BlockSpec((8, NTILES_PAD), lambda t: (0, 0)),
        ],
        out_shape=[
            jax.ShapeDtypeStruct((T, 1), jnp.int32),
            jax.ShapeDtypeStruct((T, 1), jnp.int32),
            jax.ShapeDtypeStruct((T, 1), jnp.float32),
            jax.ShapeDtypeStruct((T, 1), jnp.float32),
            jax.ShapeDtypeStruct((8, NTILES_PAD), jnp.int32),
        ],
        scratch_shapes=[
            pltpu.VMEM((1, 16), jnp.float32),
            pltpu.VMEM((T, 1), jnp.int32),
            pltpu.VMEM((T, 1), jnp.int32),
            pltpu.VMEM((T, 1), jnp.int32),
            pltpu.VMEM((T, 1), jnp.int32),
        ],
    )(x, Wg)


def _dispatch_body(x_hbm, r1_hbm, r2_hbm, xd_hbm,
                   rows_v, r1v, r2v, sa, sb, sc, sd, se):
    wid = lax.axis_index("s") * 2 + lax.axis_index("c")
    base_t = wid * TPW
    c1 = pltpu.async_copy(r1_hbm.at[pl.ds(base_t, TPW)], r1v, sa)
    c2 = pltpu.async_copy(r2_hbm.at[pl.ds(base_t, TPW)], r2v, sb)
    cx = pltpu.async_copy(x_hbm.at[pl.ds(base_t, TPW)], rows_v, sc)
    c1.wait()
    cx.wait()
    s1 = pltpu.async_copy(rows_v, xd_hbm.at[r1v], sd)
    c2.wait()
    s2 = pltpu.async_copy(rows_v, xd_hbm.at[r2v], se)
    s1.wait()
    s2.wait()


def _dispatch(x, r1, r2):
    mesh = plsc.VectorSubcoreMesh(core_axis_name="c", subcore_axis_name="s")
    fn = pl.kernel(
        _dispatch_body,
        out_type=[
            jax.ShapeDtypeStruct((R_MAX, D), jnp.float32),
        ],
        mesh=mesh,
        scratch_types=[
            pltpu.VMEM((TPW, D), jnp.float32),
            pltpu.VMEM((TPW,), jnp.int32),
            pltpu.VMEM((TPW,), jnp.int32),
            pltpu.SemaphoreType.DMA,
            pltpu.SemaphoreType.DMA,
            pltpu.SemaphoreType.DMA,
            pltpu.SemaphoreType.DMA,
            pltpu.SemaphoreType.DMA,
        ],
    )
    return fn(x, r1, r2)[0]


def _gmm_body(tmap_ref, xd_ref, w1_any, w2_any, b1_ref, b2_ref, yd_ref,
              w1b, w2b, s1, s2):
    i = pl.program_id(0)
    e = tmap_ref[0, i]
    slot = tmap_ref[1, i]
    first = tmap_ref[2, i]
    nxt = tmap_ref[3, i]

    NC = 4
    CD, CF = D // NC, F // NC

    def _chunks(dst, expert):
        for c in range(NC):
            yield pltpu.make_async_copy(
                w1_any.at[expert, pl.ds(c * CD, CD), :],
                w1b.at[dst, pl.ds(c * CD, CD), :], s1.at[dst])
            yield pltpu.make_async_copy(
                w2_any.at[expert, pl.ds(c * CF, CF), :],
                w2b.at[dst, pl.ds(c * CF, CF), :], s2.at[dst])

    def _issue(dst, expert):
        for cp in _chunks(dst, expert):
            cp.start()

    def _wait(dst, expert):
        for cp in _chunks(dst, expert):
            cp.wait()

    @pl.when(i == 0)
    def _cold():
        _issue(0, e)

    @pl.when(first == 1)
    def _boundary():
        @pl.when(jnp.logical_and(nxt < E, slot == 0))
        def _pf1():
            _issue(1, nxt)

        @pl.when(jnp.logical_and(nxt < E, slot == 1))
        def _pf0():
            _issue(0, nxt)

        @pl.when(slot == 0)
        def _w0():
            _wait(0, e)

        @pl.when(slot == 1)
        def _w1():
            _wait(1, e)

    def _compute(s):
        h = jnp.dot(xd_ref[...].astype(jnp.bfloat16),
                    w1b[s].astype(jnp.bfloat16),
                    preferred_element_type=jnp.float32)
        h = jax.nn.gelu(h + b1_ref[0])
        yd_ref[...] = jnp.dot(h.astype(jnp.bfloat16),
                              w2b[s].astype(jnp.bfloat16),
                              preferred_element_type=jnp.float32) + b2_ref[0]

    @pl.when(jnp.logical_and(e < E, slot == 0))
    def _c0():
        _compute(0)

    @pl.when(jnp.logical_and(e < E, slot == 1))
    def _c1():
        _compute(1)


def _gmm(tmap, Xd, W1, b1, W2, b2):
    def bmap(i, tm):
        return (jnp.minimum(tm[0, i], E - 1), 0, 0)

    grid_spec = pltpu.PrefetchScalarGridSpec(
        num_scalar_prefetch=1,
        grid=(NT_MAX,),
        in_specs=[
            pl.BlockSpec((TR, D), lambda i, tm: (i, 0)),
            pl.BlockSpec(memory_space=pltpu.MemorySpace.HBM),
            pl.BlockSpec(memory_space=pltpu.MemorySpace.HBM),
            pl.BlockSpec((1, 1, F), bmap),
            pl.BlockSpec((1, 1, D), bmap),
        ],
        out_specs=pl.BlockSpec((TR, D), lambda i, tm: (i, 0)),
        scratch_shapes=[
            pltpu.VMEM((2, D, F), jnp.float32),
            pltpu.VMEM((2, F, D), jnp.float32),
            pltpu.SemaphoreType.DMA((2,)),
            pltpu.SemaphoreType.DMA((2,)),
        ],
    )
    return pl.pallas_call(
        _gmm_body,
        grid_spec=grid_spec,
        out_shape=jax.ShapeDtypeStruct((R_MAX, D), jnp.float32),
        compiler_params=pltpu.CompilerParams(
            dimension_semantics=("arbitrary",),
        ),
    )(tmap, Xd, W1, W2, b1.reshape(E, 1, F), b2.reshape(E, 1, D))


def _gather2_body(yd_hbm, r1_hbm, r2_hbm, y1_hbm, y2_hbm,
                  r1v, r2v, a_v, b_v, sem1, sem2, sem3, sem4):
    wid = lax.axis_index("s") * 2 + lax.axis_index("c")
    base_t = wid * TPW
    c1 = pltpu.async_copy(r1_hbm.at[pl.ds(base_t, TPW)], r1v, sem3)
    c2 = pltpu.async_copy(r2_hbm.at[pl.ds(base_t, TPW)], r2v, sem4)
    c1.wait()
    cp1 = pltpu.async_copy(yd_hbm.at[r1v], a_v, sem1)
    c2.wait()
    cp2 = pltpu.async_copy(yd_hbm.at[r2v], b_v, sem2)
    cp1.wait()
    o1 = pltpu.async_copy(a_v, y1_hbm.at[pl.ds(base_t, TPW)], sem3)
    cp2.wait()
    o2 = pltpu.async_copy(b_v, y2_hbm.at[pl.ds(base_t, TPW)], sem4)
    o1.wait()
    o2.wait()


def _gather2(Yd, r1, r2):
    mesh = plsc.VectorSubcoreMesh(core_axis_name="c", subcore_axis_name="s")
    fn = pl.kernel(
        _gather2_body,
        out_type=[
            jax.ShapeDtypeStruct((T, D), jnp.float32),
            jax.ShapeDtypeStruct((T, D), jnp.float32),
        ],
        mesh=mesh,
        scratch_types=[
            pltpu.VMEM((TPW,), jnp.int32),
            pltpu.VMEM((TPW,), jnp.int32),
            pltpu.VMEM((TPW, D), jnp.float32),
            pltpu.VMEM((TPW, D), jnp.float32),
            pltpu.SemaphoreType.DMA,
            pltpu.SemaphoreType.DMA,
            pltpu.SemaphoreType.DMA,
            pltpu.SemaphoreType.DMA,
        ],
    )
    return fn(Yd, r1, r2)


def _fma_body(y1_ref, y2_ref, w1_ref, w2_ref, y_ref):
    y_ref[...] = w1_ref[...] * y1_ref[...] + w2_ref[...] * y2_ref[...]


def _combine(Y1, Y2, w1, w2):
    return pl.pallas_call(
        _fma_body,
        grid=(T // TT,),
        in_specs=[
            pl.BlockSpec((TT, D), lambda t: (t, 0)),
            pl.BlockSpec((TT, D), lambda t: (t, 0)),
            pl.BlockSpec((TT, 1), lambda t: (t, 0)),
            pl.BlockSpec((TT, 1), lambda t: (t, 0)),
        ],
        out_specs=pl.BlockSpec((TT, D), lambda t: (t, 0)),
        out_shape=jax.ShapeDtypeStruct((T, D), jnp.float32),
    )(Y1, Y2, w1, w2)


def kernel(x, Wg, W1, b1, W2, b2):
    r1, r2, w1, w2, tmap = _gate(x, Wg)
    r1 = r1.reshape(T)
    r2 = r2.reshape(T)
    Xd = _dispatch(x, r1, r2)
    Yd = _gmm(tmap, Xd, W1, b1, W2, b2)
    Y1, Y2 = _gather2(Yd, r1, r2)
    return _combine(Y1, Y2, w1, w2)
